# baseline (speedup 1.0000x reference)
import jax
import jax.numpy as jnp
from jax.experimental import pallas as pl
from jax.experimental.pallas import tpu as pltpu


def kernel(x, A, W, b, gamma, beta):
    del b
    n, d_in = x.shape
    d_out = W.shape[1]
    BM = 200
    M = n // BM
    eps = 1e-5

    def body(x_ref, a_ref, w_ref, g_ref, bta_ref, out_ref, p_ref, h_ref, s_ref):
        g = pl.program_id(0)

        @pl.when(g == 0)
        def _():
            p_ref[...] = jnp.dot(x_ref[...], w_ref[...],
                                 preferred_element_type=jnp.float32)
            s_ref[...] = jnp.zeros_like(s_ref)

        @pl.when((g >= 1) & (g <= M))
        def _():
            h = jnp.dot(a_ref[...], p_ref[...],
                        preferred_element_type=jnp.float32)
            h_ref[pl.ds((g - 1) * BM, BM), :] = h
            s_ref[0:1, :] += jnp.sum(h, axis=0, keepdims=True)
            s_ref[1:2, :] += jnp.sum(h * h, axis=0, keepdims=True)

        @pl.when(g > M)
        def _():
            i = g - M - 1
            h = h_ref[pl.ds(i * BM, BM), :]
            mean = s_ref[0:1, :] * (1.0 / n)
            var = s_ref[1:2, :] * (1.0 / n) - mean * mean
            scale = jax.lax.rsqrt(var + eps) * g_ref[...]
            y = (h - mean) * scale + bta_ref[...]
            out_ref[...] = jnp.where(y >= 0, y, 0.2 * y)

    out = pl.pallas_call(
        body,
        grid=(2 * M + 1,),
        in_specs=[
            pl.BlockSpec((n, d_in), lambda g: (0, 0)),
            pl.BlockSpec((BM, n),
                         lambda g: (jnp.clip(g - 1, 0, M - 1), 0)),
            pl.BlockSpec((d_in, d_out), lambda g: (0, 0)),
            pl.BlockSpec((1, d_out), lambda g: (0, 0)),
            pl.BlockSpec((1, d_out), lambda g: (0, 0)),
        ],
        out_specs=pl.BlockSpec((BM, d_out),
                               lambda g: (jnp.maximum(g - M - 1, 0), 0)),
        out_shape=jax.ShapeDtypeStruct((n, d_out), jnp.float32),
        scratch_shapes=[
            pltpu.VMEM((n, d_out), jnp.float32),
            pltpu.VMEM((n, d_out), jnp.float32),
            pltpu.VMEM((8, d_out), jnp.float32),
        ],
        compiler_params=pltpu.CompilerParams(
            dimension_semantics=("arbitrary",),
            vmem_limit_bytes=100 * 1024 * 1024,
        ),
    )(x, A, W, gamma.reshape(1, -1), beta.reshape(1, -1))
    return out


# speedup vs baseline: 1.0965x; 1.0965x over previous
import jax
import jax.numpy as jnp
from jax.experimental import pallas as pl
from jax.experimental.pallas import tpu as pltpu


def kernel(x, A, W, b, gamma, beta):
    del b
    n, d_in = x.shape
    d_out = W.shape[1]
    BM = 400
    BM2 = 1000
    M = n // BM
    M2 = n // BM2
    eps = 1e-5

    def body(x_ref, a_ref, w_ref, g_ref, bta_ref, out_ref, p_ref, h_ref, s_ref):
        g = pl.program_id(0)

        @pl.when(g == 0)
        def _():
            p_ref[...] = jnp.dot(x_ref[...], w_ref[...],
                                 preferred_element_type=jnp.float32)
            s_ref[...] = jnp.zeros_like(s_ref)

        @pl.when((g >= 1) & (g <= M))
        def _():
            h = jnp.dot(a_ref[...], p_ref[...],
                        preferred_element_type=jnp.float32)
            h_ref[pl.ds((g - 1) * BM, BM), :] = h
            s_ref[0:1, :] += jnp.sum(h, axis=0, keepdims=True)
            s_ref[1:2, :] += jnp.sum(h * h, axis=0, keepdims=True)

        @pl.when(g > M)
        def _():
            j = g - M - 1
            h = h_ref[pl.ds(j * BM2, BM2), :]
            mean = s_ref[0:1, :] * (1.0 / n)
            var = s_ref[1:2, :] * (1.0 / n) - mean * mean
            scale = jax.lax.rsqrt(var + eps) * g_ref[...]
            y = (h - mean) * scale + bta_ref[...]
            out_ref[...] = jnp.where(y >= 0, y, 0.2 * y)

    out = pl.pallas_call(
        body,
        grid=(1 + M + M2,),
        in_specs=[
            pl.BlockSpec((n, d_in), lambda g: (0, 0)),
            pl.BlockSpec((BM, n),
                         lambda g: (jnp.clip(g - 1, 0, M - 1), 0)),
            pl.BlockSpec((d_in, d_out), lambda g: (0, 0)),
            pl.BlockSpec((1, d_out), lambda g: (0, 0)),
            pl.BlockSpec((1, d_out), lambda g: (0, 0)),
        ],
        out_specs=pl.BlockSpec((BM2, d_out),
                               lambda g: (jnp.maximum(g - M - 1, 0), 0)),
        out_shape=jax.ShapeDtypeStruct((n, d_out), jnp.float32),
        scratch_shapes=[
            pltpu.VMEM((n, d_out), jnp.float32),
            pltpu.VMEM((n, d_out), jnp.float32),
            pltpu.VMEM((8, d_out), jnp.float32),
        ],
        compiler_params=pltpu.CompilerParams(
            dimension_semantics=("arbitrary",),
            vmem_limit_bytes=100 * 1024 * 1024,
        ),
    )(x, A, W, gamma.reshape(1, -1), beta.reshape(1, -1))
    return out


# speedup vs baseline: 1.1234x; 1.0246x over previous
import jax
import jax.numpy as jnp
from jax.experimental import pallas as pl
from jax.experimental.pallas import tpu as pltpu


def kernel(x, A, W, b, gamma, beta):
    del b
    n, d_in = x.shape
    d_out = W.shape[1]
    BM = 400
    M = n // BM
    eps = 1e-5

    def body(x_ref, a_ref, w_ref, g_ref, bta_ref, out_ref, h_ref, s_ref):
        g = pl.program_id(0)

        @pl.when(g == 0)
        def _():
            s_ref[...] = jnp.zeros_like(s_ref)

        @pl.when(g < M)
        def _():
            ax = jnp.dot(a_ref[...], x_ref[...],
                         preferred_element_type=jnp.float32)
            h = jnp.dot(ax, w_ref[...], preferred_element_type=jnp.float32)
            h_ref[pl.ds(g * BM, BM), :] = h
            s_ref[0:1, :] += jnp.sum(h, axis=0, keepdims=True)
            s_ref[1:2, :] += jnp.sum(h * h, axis=0, keepdims=True)

        @pl.when(g == M)
        def _():
            h = h_ref[...]
            mean = s_ref[0:1, :] * (1.0 / n)
            var = s_ref[1:2, :] * (1.0 / n) - mean * mean
            scale = jax.lax.rsqrt(var + eps) * g_ref[...]
            y = (h - mean) * scale + bta_ref[...]
            out_ref[...] = jnp.where(y >= 0, y, 0.2 * y)

    out = pl.pallas_call(
        body,
        grid=(M + 1,),
        in_specs=[
            pl.BlockSpec((n, d_in), lambda g: (0, 0)),
            pl.BlockSpec((BM, n),
                         lambda g: (jnp.minimum(g, M - 1), 0)),
            pl.BlockSpec((d_in, d_out), lambda g: (0, 0)),
            pl.BlockSpec((1, d_out), lambda g: (0, 0)),
            pl.BlockSpec((1, d_out), lambda g: (0, 0)),
        ],
        out_specs=pl.BlockSpec((n, d_out), lambda g: (0, 0)),
        out_shape=jax.ShapeDtypeStruct((n, d_out), jnp.float32),
        scratch_shapes=[
            pltpu.VMEM((n, d_out), jnp.float32),
            pltpu.VMEM((8, d_out), jnp.float32),
        ],
        compiler_params=pltpu.CompilerParams(
            dimension_semantics=("arbitrary",),
            vmem_limit_bytes=100 * 1024 * 1024,
        ),
    )(x, A, W, gamma.reshape(1, -1), beta.reshape(1, -1))
    return out
